# Initial kernel scaffold; baseline (speedup 1.0000x reference)
import functools

import jax
import jax.numpy as jnp
from jax import lax
from jax.experimental import pallas as pl
from jax.experimental.pallas import tpu as pltpu
from jax.experimental.pallas import tpu_sc as plsc

N = 10000
E = 320000
D_IN = 128
HID = 128
H = 8
D_OUT = 128
G = 64
NEG_SLOPE = 0.2

NC = 2
NS = 16
NW = NC * NS
E_PER_W = E // NW

_mesh = plsc.VectorSubcoreMesh(
    core_axis_name="c", subcore_axis_name="s", num_cores=NC, num_subcores=NS
)


def _k1_body(x_ref, wl_ref, bl_ref, wr_ref, br_ref, xl_ref, xr_ref):
    x = x_ref[...]
    xl_ref[...] = jnp.dot(x, wl_ref[...], preferred_element_type=jnp.float32) + bl_ref[...]
    xr_ref[...] = jnp.dot(x, wr_ref[...], preferred_element_type=jnp.float32) + br_ref[...]


def _k1(x, W_l, b_l, W_r, b_r):
    R = 1000
    return pl.pallas_call(
        _k1_body,
        grid=(N // R,),
        in_specs=[
            pl.BlockSpec((R, D_IN), lambda i: (i, 0)),
            pl.BlockSpec((D_IN, H * HID), lambda i: (0, 0)),
            pl.BlockSpec((1, H * HID), lambda i: (0, 0)),
            pl.BlockSpec((D_IN, H * HID), lambda i: (0, 0)),
            pl.BlockSpec((1, H * HID), lambda i: (0, 0)),
        ],
        out_specs=[
            pl.BlockSpec((R, H * HID), lambda i: (i, 0)),
            pl.BlockSpec((R, H * HID), lambda i: (i, 0)),
        ],
        out_shape=[
            jax.ShapeDtypeStruct((N, H * HID), jnp.float32),
            jax.ShapeDtypeStruct((N, H * HID), jnp.float32),
        ],
    )(x, W_l, b_l.reshape(1, -1), W_r, b_r.reshape(1, -1))


_K2_C = 40


def _k2_body(xl_hbm, xr_hbm, src_hbm, dst_hbm, xj_hbm, xi_hbm,
             ia_ref, ib_ref, a_ref, b_ref):
    wid = lax.axis_index("c") * NS + lax.axis_index("s")
    base = wid * E_PER_W

    @pl.loop(0, E_PER_W // _K2_C)
    def _(i):
        e0 = base + i * _K2_C
        pltpu.sync_copy(src_hbm.at[pl.ds(e0, _K2_C)], ia_ref)
        pltpu.sync_copy(dst_hbm.at[pl.ds(e0, _K2_C)], ib_ref)
        pltpu.sync_copy(xl_hbm.at[ia_ref], a_ref)
        pltpu.sync_copy(xr_hbm.at[ib_ref], b_ref)
        pltpu.sync_copy(a_ref, xj_hbm.at[pl.ds(e0, _K2_C)])
        pltpu.sync_copy(b_ref, xi_hbm.at[pl.ds(e0, _K2_C)])


def _k2(xl, xr, src, dst):
    f = pl.kernel(
        _k2_body,
        out_type=[
            jax.ShapeDtypeStruct((E, H * HID), jnp.float32),
            jax.ShapeDtypeStruct((E, H * HID), jnp.float32),
        ],
        mesh=_mesh,
        scratch_types=[
            pltpu.VMEM((_K2_C,), jnp.int32),
            pltpu.VMEM((_K2_C,), jnp.int32),
            pltpu.VMEM((_K2_C, H * HID), jnp.float32),
            pltpu.VMEM((_K2_C, H * HID), jnp.float32),
        ],
    )
    return f(xl, xr, src, dst)


def _k3_body(xj_ref, xi_ref, ef_ref, we_ref, att_ref, ex_ref, *msg_refs):
    ee = jnp.dot(ef_ref[...], we_ref[...], preferred_element_type=jnp.float32)
    u = xj_ref[...] + xi_ref[...] + ee
    z = jnp.where(u >= 0.0, u, NEG_SLOPE * u)
    att = att_ref[...]
    xj = xj_ref[...]
    exs = []
    for h in range(H):
        zh = z[:, h * HID:(h + 1) * HID]
        alpha_h = jnp.sum(zh * att[h:h + 1, :], axis=1, keepdims=True)
        ex_h = jnp.exp(alpha_h)
        exs.append(ex_h)
        msg_refs[h][...] = xj[:, h * HID:(h + 1) * HID] * ex_h
    ex = jnp.concatenate(exs, axis=1)
    ex_ref[...] = jnp.concatenate([ex, jnp.zeros_like(ex)], axis=1)


def _k3(xj, xi, ef, W_e, att):
    BE = 1000
    return pl.pallas_call(
        _k3_body,
        grid=(E // BE,),
        in_specs=[
            pl.BlockSpec((BE, H * HID), lambda i: (i, 0)),
            pl.BlockSpec((BE, H * HID), lambda i: (i, 0)),
            pl.BlockSpec((BE, D_IN), lambda i: (i, 0)),
            pl.BlockSpec((D_IN, H * HID), lambda i: (0, 0)),
            pl.BlockSpec((H, HID), lambda i: (0, 0)),
        ],
        out_specs=[pl.BlockSpec((BE, 16), lambda i: (i, 0))]
        + [pl.BlockSpec((BE, HID), lambda i: (i, 0)) for _ in range(H)],
        out_shape=[jax.ShapeDtypeStruct((E, 16), jnp.float32)]
        + [jax.ShapeDtypeStruct((E, HID), jnp.float32) for _ in range(H)],
    )(xj, xi, ef, W_e, att)


_K4_C = 80


def _k4_body(ex_hbm, dst_hbm, z_hbm, dpart_hbm, buf_ref, idx_ref, acc_shared):
    cid = lax.axis_index("c")
    sid = lax.axis_index("s")
    wid = cid * NS + sid
    base = wid * E_PER_W

    @pl.when(sid == 0)
    def _():
        pltpu.sync_copy(z_hbm, acc_shared)

    plsc.subcore_barrier()

    @pl.loop(0, E_PER_W // _K4_C)
    def _(i):
        e0 = base + i * _K4_C
        pltpu.sync_copy(ex_hbm.at[pl.ds(e0, _K4_C)], buf_ref)
        pltpu.sync_copy(dst_hbm.at[pl.ds(e0, _K4_C)], idx_ref)
        pltpu.sync_copy(buf_ref, acc_shared.at[idx_ref], add=True)

    plsc.subcore_barrier()

    @pl.when(sid == 0)
    def _():
        pltpu.sync_copy(acc_shared, dpart_hbm.at[cid])


def _k4(ex, dst, zeros16):
    f = pl.kernel(
        _k4_body,
        out_type=jax.ShapeDtypeStruct((NC, N, 16), jnp.float32),
        mesh=_mesh,
        scratch_types=[
            pltpu.VMEM((_K4_C, 16), jnp.float32),
            pltpu.VMEM((_K4_C,), jnp.int32),
            pltpu.VMEM_SHARED((N, 16), jnp.float32),
        ],
    )
    return f(ex, dst, zeros16)


_K5_C = 80


def _k5_body(m0, m1, m2, m3, m4, m5, m6, m7, dst_hbm, z_hbm, mpart_hbm,
             buf_ref, idx_ref, acc_shared):
    cid = lax.axis_index("c")
    sid = lax.axis_index("s")
    wid = cid * NS + sid
    base = wid * E_PER_W
    msgs = (m0, m1, m2, m3, m4, m5, m6, m7)

    for h in range(H):
        @pl.when(sid == 0)
        def _():
            pltpu.sync_copy(z_hbm, acc_shared)

        plsc.subcore_barrier()

        @pl.loop(0, E_PER_W // _K5_C)
        def _(i):
            e0 = base + i * _K5_C
            pltpu.sync_copy(msgs[h].at[pl.ds(e0, _K5_C)], buf_ref)
            pltpu.sync_copy(dst_hbm.at[pl.ds(e0, _K5_C)], idx_ref)
            pltpu.sync_copy(buf_ref, acc_shared.at[idx_ref], add=True)

        plsc.subcore_barrier()

        @pl.when(sid == 0)
        def _():
            pltpu.sync_copy(acc_shared, mpart_hbm.at[cid, h])

        plsc.subcore_barrier()


def _k5(msgs, dst, zerosN):
    f = pl.kernel(
        _k5_body,
        out_type=jax.ShapeDtypeStruct((NC, H, N, HID), jnp.float32),
        mesh=_mesh,
        scratch_types=[
            pltpu.VMEM((_K5_C, HID), jnp.float32),
            pltpu.VMEM((_K5_C,), jnp.int32),
            pltpu.VMEM_SHARED((N, HID), jnp.float32),
        ],
    )
    return f(*msgs, dst, zerosN)


def _k6_body(mpart_ref, dpart_ref, bias_ref, out_ref):
    m = mpart_ref[...]
    d = dpart_ref[...]
    den = d[0] + d[1]
    acc = None
    for h in range(H):
        wh = 1.0 / (den[:, h:h + 1] + 1e-16)
        t = (m[0, h] + m[1, h]) * wh
        acc = t if acc is None else acc + t
    out_ref[...] = jnp.tanh(acc * (1.0 / H) + bias_ref[...])


def _k6(mpart, dpart, bias):
    R = 1000
    return pl.pallas_call(
        _k6_body,
        grid=(N // R,),
        in_specs=[
            pl.BlockSpec((NC, H, R, HID), lambda i: (0, 0, i, 0)),
            pl.BlockSpec((NC, R, 16), lambda i: (0, i, 0)),
            pl.BlockSpec((1, HID), lambda i: (0, 0)),
        ],
        out_specs=pl.BlockSpec((R, HID), lambda i: (i, 0)),
        out_shape=jax.ShapeDtypeStruct((N, HID), jnp.float32),
    )(mpart, dpart, bias.reshape(1, -1))


def _k7_body(h_ref, batch_ref, wlin_ref, blin_ref, out_ref):
    b = batch_ref[...]
    gid = lax.broadcasted_iota(jnp.int32, (G, N), 0)
    onehot = (gid == b).astype(jnp.float32)
    counts = jnp.sum(onehot, axis=1, keepdims=True)
    gsum = jnp.dot(onehot, h_ref[...], preferred_element_type=jnp.float32)
    gmean = gsum / jnp.maximum(counts, 1.0)
    out_ref[...] = jnp.tanh(
        jnp.dot(gmean, wlin_ref[...], preferred_element_type=jnp.float32)
        + blin_ref[...]
    )


def _k7(h_nodes, batch, W_lin, b_lin):
    return pl.pallas_call(
        _k7_body,
        grid=(1,),
        in_specs=[
            pl.BlockSpec((N, HID), lambda i: (0, 0)),
            pl.BlockSpec((1, N), lambda i: (0, 0)),
            pl.BlockSpec((HID, D_OUT), lambda i: (0, 0)),
            pl.BlockSpec((1, D_OUT), lambda i: (0, 0)),
        ],
        out_specs=pl.BlockSpec((G, D_OUT), lambda i: (0, 0)),
        out_shape=jax.ShapeDtypeStruct((G, D_OUT), jnp.float32),
    )(h_nodes, batch.reshape(1, -1), W_lin, b_lin.reshape(1, -1))


def kernel(x, edge_index, edge_features, batch, W_l, b_l, W_r, b_r, W_e, att,
           bias, W_lin, b_lin):
    src = edge_index[0]
    dst = edge_index[1]
    xl, xr = _k1(x, W_l, b_l, W_r, b_r)
    xj, xi = _k2(xl, xr, src, dst)
    ex, *msgs = _k3(xj, xi, edge_features, W_e, att)
    dpart = _k4(ex, dst, jnp.zeros((N, 16), jnp.float32))
    mpart = _k5(msgs, dst, jnp.zeros((N, HID), jnp.float32))
    h_nodes = _k6(mpart, dpart, bias)
    graph_embeddings = _k7(h_nodes, batch, W_lin, b_lin)
    return graph_embeddings, h_nodes


# speedup vs baseline: 7.5418x; 7.5418x over previous
import functools

import jax
import jax.numpy as jnp
from jax import lax
from jax.experimental import pallas as pl
from jax.experimental.pallas import tpu as pltpu
from jax.experimental.pallas import tpu_sc as plsc

N = 10000
E = 320000
D_IN = 128
HID = 128
H = 8
D_OUT = 128
G = 64
NEG_SLOPE = 0.2

NC = 2
NS = 16
NW = NC * NS
E_PER_W = E // NW

_mesh = plsc.VectorSubcoreMesh(
    core_axis_name="c", subcore_axis_name="s", num_cores=NC, num_subcores=NS
)


def _k1_body(x_ref, wl_ref, bl_ref, wr_ref, br_ref, xl_ref, xr_ref):
    x = x_ref[...]
    xl_ref[...] = jnp.dot(x, wl_ref[...], preferred_element_type=jnp.float32) + bl_ref[...]
    xr_ref[...] = jnp.dot(x, wr_ref[...], preferred_element_type=jnp.float32) + br_ref[...]


def _k1(x, W_l, b_l, W_r, b_r):
    R = 1000
    return pl.pallas_call(
        _k1_body,
        grid=(N // R,),
        in_specs=[
            pl.BlockSpec((R, D_IN), lambda i: (i, 0)),
            pl.BlockSpec((D_IN, H * HID), lambda i: (0, 0)),
            pl.BlockSpec((1, H * HID), lambda i: (0, 0)),
            pl.BlockSpec((D_IN, H * HID), lambda i: (0, 0)),
            pl.BlockSpec((1, H * HID), lambda i: (0, 0)),
        ],
        out_specs=[
            pl.BlockSpec((R, H * HID), lambda i: (i, 0)),
            pl.BlockSpec((R, H * HID), lambda i: (i, 0)),
        ],
        out_shape=[
            jax.ShapeDtypeStruct((N, H * HID), jnp.float32),
            jax.ShapeDtypeStruct((N, H * HID), jnp.float32),
        ],
    )(x, W_l, b_l.reshape(1, -1), W_r, b_r.reshape(1, -1))


_K2_C = 40


def _k2_body(xl_hbm, xr_hbm, src_hbm, dst_hbm, xj_hbm, xi_hbm,
             ia_ref, ib_ref, a_ref, b_ref):
    wid = lax.axis_index("c") * NS + lax.axis_index("s")
    base = wid * E_PER_W

    @pl.loop(0, E_PER_W // _K2_C)
    def _(i):
        e0 = base + i * _K2_C
        pltpu.sync_copy(src_hbm.at[pl.ds(e0, _K2_C)], ia_ref)
        pltpu.sync_copy(dst_hbm.at[pl.ds(e0, _K2_C)], ib_ref)
        pltpu.sync_copy(xl_hbm.at[ia_ref], a_ref)
        pltpu.sync_copy(xr_hbm.at[ib_ref], b_ref)
        pltpu.sync_copy(a_ref, xj_hbm.at[pl.ds(e0, _K2_C)])
        pltpu.sync_copy(b_ref, xi_hbm.at[pl.ds(e0, _K2_C)])


def _k2(xl, xr, src, dst):
    f = pl.kernel(
        _k2_body,
        out_type=[
            jax.ShapeDtypeStruct((E, H * HID), jnp.float32),
            jax.ShapeDtypeStruct((E, H * HID), jnp.float32),
        ],
        mesh=_mesh,
        scratch_types=[
            pltpu.VMEM((_K2_C,), jnp.int32),
            pltpu.VMEM((_K2_C,), jnp.int32),
            pltpu.VMEM((_K2_C, H * HID), jnp.float32),
            pltpu.VMEM((_K2_C, H * HID), jnp.float32),
        ],
    )
    return f(xl, xr, src, dst)


def _k3_body(xj_ref, xi_ref, ef_ref, we_ref, att_ref, ex_ref, *msg_refs):
    ee = jnp.dot(ef_ref[...], we_ref[...], preferred_element_type=jnp.float32)
    u = xj_ref[...] + xi_ref[...] + ee
    z = jnp.where(u >= 0.0, u, NEG_SLOPE * u)
    att = att_ref[...]
    xj = xj_ref[...]
    exs = []
    for h in range(H):
        zh = z[:, h * HID:(h + 1) * HID]
        alpha_h = jnp.sum(zh * att[h:h + 1, :], axis=1, keepdims=True)
        ex_h = jnp.exp(alpha_h)
        exs.append(ex_h)
        msg_refs[h][...] = xj[:, h * HID:(h + 1) * HID] * ex_h
    ex = jnp.concatenate(exs, axis=1)
    ex_ref[...] = jnp.concatenate(
        [ex, jnp.zeros((ex.shape[0], HID - H), jnp.float32)], axis=1)


def _k3(xj, xi, ef, W_e, att):
    BE = 1000
    return pl.pallas_call(
        _k3_body,
        grid=(E // BE,),
        in_specs=[
            pl.BlockSpec((BE, H * HID), lambda i: (i, 0)),
            pl.BlockSpec((BE, H * HID), lambda i: (i, 0)),
            pl.BlockSpec((BE, D_IN), lambda i: (i, 0)),
            pl.BlockSpec((D_IN, H * HID), lambda i: (0, 0)),
            pl.BlockSpec((H, HID), lambda i: (0, 0)),
        ],
        out_specs=[pl.BlockSpec((BE, HID), lambda i: (i, 0))]
        + [pl.BlockSpec((BE, HID), lambda i: (i, 0)) for _ in range(H)],
        out_shape=[jax.ShapeDtypeStruct((E, HID), jnp.float32)]
        + [jax.ShapeDtypeStruct((E, HID), jnp.float32) for _ in range(H)],
    )(xj, xi, ef, W_e, att)


_K5_C = 80
_NCH = H + 1


def _k5_body(m0, m1, m2, m3, m4, m5, m6, m7, m8, dst_hbm, z_hbm, mpart_hbm,
             buf_ref, idx_ref, acc_shared):
    cid = lax.axis_index("c")
    sid = lax.axis_index("s")
    wid = cid * NS + sid
    base = wid * E_PER_W
    msgs = (m0, m1, m2, m3, m4, m5, m6, m7, m8)

    for h in range(_NCH):
        @pl.when(sid == 0)
        def _():
            pltpu.sync_copy(z_hbm, acc_shared)

        plsc.subcore_barrier()

        @pl.loop(0, E_PER_W // _K5_C)
        def _(i):
            e0 = base + i * _K5_C
            pltpu.sync_copy(msgs[h].at[pl.ds(e0, _K5_C)], buf_ref)
            pltpu.sync_copy(dst_hbm.at[pl.ds(e0, _K5_C)], idx_ref)
            pltpu.sync_copy(buf_ref, acc_shared.at[idx_ref], add=True)

        plsc.subcore_barrier()

        @pl.when(sid == 0)
        def _():
            pltpu.sync_copy(acc_shared, mpart_hbm.at[cid, h])

        plsc.subcore_barrier()


def _k5(msgs, dst, zerosN):
    f = pl.kernel(
        _k5_body,
        out_type=jax.ShapeDtypeStruct((NC, _NCH, N, HID), jnp.float32),
        mesh=_mesh,
        scratch_types=[
            pltpu.VMEM((_K5_C, HID), jnp.float32),
            pltpu.VMEM((_K5_C,), jnp.int32),
            pltpu.VMEM_SHARED((N, HID), jnp.float32),
        ],
    )
    return f(*msgs, dst, zerosN)


def _k6_body(mpart_ref, bias_ref, out_ref):
    m = mpart_ref[...]
    den = m[0, H] + m[1, H]
    acc = None
    for h in range(H):
        wh = 1.0 / (den[:, h:h + 1] + 1e-16)
        t = (m[0, h] + m[1, h]) * wh
        acc = t if acc is None else acc + t
    out_ref[...] = jnp.tanh(acc * (1.0 / H) + bias_ref[...])


def _k6(mpart, bias):
    R = 1000
    return pl.pallas_call(
        _k6_body,
        grid=(N // R,),
        in_specs=[
            pl.BlockSpec((NC, _NCH, R, HID), lambda i: (0, 0, i, 0)),
            pl.BlockSpec((1, HID), lambda i: (0, 0)),
        ],
        out_specs=pl.BlockSpec((R, HID), lambda i: (i, 0)),
        out_shape=jax.ShapeDtypeStruct((N, HID), jnp.float32),
    )(mpart, bias.reshape(1, -1))


def _k7_body(h_ref, batch_ref, wlin_ref, blin_ref, out_ref):
    b = batch_ref[...]
    gid = lax.broadcasted_iota(jnp.int32, (G, N), 0)
    onehot = (gid == b).astype(jnp.float32)
    counts = jnp.sum(onehot, axis=1, keepdims=True)
    gsum = jnp.dot(onehot, h_ref[...], preferred_element_type=jnp.float32)
    gmean = gsum / jnp.maximum(counts, 1.0)
    out_ref[...] = jnp.tanh(
        jnp.dot(gmean, wlin_ref[...], preferred_element_type=jnp.float32)
        + blin_ref[...]
    )


def _k7(h_nodes, batch, W_lin, b_lin):
    return pl.pallas_call(
        _k7_body,
        grid=(1,),
        in_specs=[
            pl.BlockSpec((N, HID), lambda i: (0, 0)),
            pl.BlockSpec((1, N), lambda i: (0, 0)),
            pl.BlockSpec((HID, D_OUT), lambda i: (0, 0)),
            pl.BlockSpec((1, D_OUT), lambda i: (0, 0)),
        ],
        out_specs=pl.BlockSpec((G, D_OUT), lambda i: (0, 0)),
        out_shape=jax.ShapeDtypeStruct((G, D_OUT), jnp.float32),
    )(h_nodes, batch.reshape(1, -1), W_lin, b_lin.reshape(1, -1))


def kernel(x, edge_index, edge_features, batch, W_l, b_l, W_r, b_r, W_e, att,
           bias, W_lin, b_lin):
    src = edge_index[0]
    dst = edge_index[1]
    xl, xr = _k1(x, W_l, b_l, W_r, b_r)
    xj, xi = _k2(xl, xr, src, dst)
    ex, *msgs = _k3(xj, xi, edge_features, W_e, att)
    mpart = _k5(msgs + [ex], dst, jnp.zeros((N, HID), jnp.float32))
    h_nodes = _k6(mpart, bias)
    graph_embeddings = _k7(h_nodes, batch, W_lin, b_lin)
    return graph_embeddings, h_nodes


# speedup vs baseline: 9.9731x; 1.3224x over previous
import functools

import jax
import jax.numpy as jnp
from jax import lax
from jax.experimental import pallas as pl
from jax.experimental.pallas import tpu as pltpu
from jax.experimental.pallas import tpu_sc as plsc

N = 10000
E = 320000
D_IN = 128
HID = 128
H = 8
D_OUT = 128
G = 64
NEG_SLOPE = 0.2

NC = 2
NS = 16
NW = NC * NS
E_PER_W = E // NW

_mesh = plsc.VectorSubcoreMesh(
    core_axis_name="c", subcore_axis_name="s", num_cores=NC, num_subcores=NS
)


def _pack_bf16(v):
    bits = lax.bitcast_convert_type(v, jnp.int32)
    r16 = (bits + 0x7FFF + ((bits >> 16) & 1)) >> 16
    a = r16[:, :512]
    b = r16[:, 512:]
    return (b << 16) | (a & 0xFFFF)


def _unpack_bf16(p):
    lo = lax.bitcast_convert_type(p << 16, jnp.float32)
    hi = lax.bitcast_convert_type(p & jnp.int32(-65536), jnp.float32)
    return jnp.concatenate([lo, hi], axis=1)


def _k1_body(x_ref, wl_ref, bl_ref, wr_ref, br_ref, xl_ref, xr_ref):
    x = x_ref[...].astype(jnp.bfloat16)
    xl = jnp.dot(x, wl_ref[...].astype(jnp.bfloat16),
                 preferred_element_type=jnp.float32) + bl_ref[...]
    xr = jnp.dot(x, wr_ref[...].astype(jnp.bfloat16),
                 preferred_element_type=jnp.float32) + br_ref[...]
    xl_ref[...] = _pack_bf16(xl)
    xr_ref[...] = _pack_bf16(xr)


def _k1(x, W_l, b_l, W_r, b_r):
    R = 1000
    return pl.pallas_call(
        _k1_body,
        grid=(N // R,),
        in_specs=[
            pl.BlockSpec((R, D_IN), lambda i: (i, 0)),
            pl.BlockSpec((D_IN, H * HID), lambda i: (0, 0)),
            pl.BlockSpec((1, H * HID), lambda i: (0, 0)),
            pl.BlockSpec((D_IN, H * HID), lambda i: (0, 0)),
            pl.BlockSpec((1, H * HID), lambda i: (0, 0)),
        ],
        out_specs=[
            pl.BlockSpec((R, H * HID // 2), lambda i: (i, 0)),
            pl.BlockSpec((R, H * HID // 2), lambda i: (i, 0)),
        ],
        out_shape=[
            jax.ShapeDtypeStruct((N, H * HID // 2), jnp.int32),
            jax.ShapeDtypeStruct((N, H * HID // 2), jnp.int32),
        ],
    )(x, W_l, b_l.reshape(1, -1), W_r, b_r.reshape(1, -1))


_K2_C = 80


def _k2_body(xl_hbm, xr_hbm, src_hbm, dst_hbm, xj_hbm, xi_hbm,
             ia_ref, ib_ref, a_ref, b_ref):
    wid = lax.axis_index("c") * NS + lax.axis_index("s")
    base = wid * E_PER_W

    @pl.loop(0, E_PER_W // _K2_C)
    def _(i):
        e0 = base + i * _K2_C
        pltpu.sync_copy(src_hbm.at[pl.ds(e0, _K2_C)], ia_ref)
        pltpu.sync_copy(dst_hbm.at[pl.ds(e0, _K2_C)], ib_ref)
        pltpu.sync_copy(xl_hbm.at[ia_ref], a_ref)
        pltpu.sync_copy(xr_hbm.at[ib_ref], b_ref)
        pltpu.sync_copy(a_ref, xj_hbm.at[pl.ds(e0, _K2_C)])
        pltpu.sync_copy(b_ref, xi_hbm.at[pl.ds(e0, _K2_C)])


def _k2(xl, xr, src, dst):
    f = pl.kernel(
        _k2_body,
        out_type=[
            jax.ShapeDtypeStruct((E, H * HID // 2), jnp.int32),
            jax.ShapeDtypeStruct((E, H * HID // 2), jnp.int32),
        ],
        mesh=_mesh,
        scratch_types=[
            pltpu.VMEM((_K2_C,), jnp.int32),
            pltpu.VMEM((_K2_C,), jnp.int32),
            pltpu.VMEM((_K2_C, H * HID // 2), jnp.int32),
            pltpu.VMEM((_K2_C, H * HID // 2), jnp.int32),
        ],
    )
    return f(xl, xr, src, dst)


def _k3_body(xj_ref, xi_ref, ef_ref, we_ref, att_ref, ex_ref, *msg_refs):
    ee = jnp.dot(ef_ref[...].astype(jnp.bfloat16),
                 we_ref[...].astype(jnp.bfloat16),
                 preferred_element_type=jnp.float32)
    xj = _unpack_bf16(xj_ref[...])
    u = xj + _unpack_bf16(xi_ref[...]) + ee
    z = jnp.where(u >= 0.0, u, NEG_SLOPE * u)
    att = att_ref[...]
    exs = []
    for h in range(H):
        zh = z[:, h * HID:(h + 1) * HID]
        alpha_h = jnp.sum(zh * att[h:h + 1, :], axis=1, keepdims=True)
        ex_h = jnp.exp(alpha_h)
        exs.append(ex_h)
        msg_refs[h][...] = xj[:, h * HID:(h + 1) * HID] * ex_h
    ex = jnp.concatenate(exs, axis=1)
    ex_ref[...] = jnp.concatenate(
        [ex, jnp.zeros((ex.shape[0], HID - H), jnp.float32)], axis=1)


def _k3(xj, xi, ef, W_e, att):
    BE = 1000
    return pl.pallas_call(
        _k3_body,
        grid=(E // BE,),
        in_specs=[
            pl.BlockSpec((BE, H * HID // 2), lambda i: (i, 0)),
            pl.BlockSpec((BE, H * HID // 2), lambda i: (i, 0)),
            pl.BlockSpec((BE, D_IN), lambda i: (i, 0)),
            pl.BlockSpec((D_IN, H * HID), lambda i: (0, 0)),
            pl.BlockSpec((H, HID), lambda i: (0, 0)),
        ],
        out_specs=[pl.BlockSpec((BE, HID), lambda i: (i, 0))]
        + [pl.BlockSpec((BE, HID), lambda i: (i, 0)) for _ in range(H)],
        out_shape=[jax.ShapeDtypeStruct((E, HID), jnp.float32)]
        + [jax.ShapeDtypeStruct((E, HID), jnp.float32) for _ in range(H)],
    )(xj, xi, ef, W_e, att)


_K5_C = 80
_NCH = H + 1


def _k5_body(m0, m1, m2, m3, m4, m5, m6, m7, m8, dst_hbm, z_hbm, mpart_hbm,
             buf_ref, idx_ref, acc_shared):
    cid = lax.axis_index("c")
    sid = lax.axis_index("s")
    wid = cid * NS + sid
    base = wid * E_PER_W
    msgs = (m0, m1, m2, m3, m4, m5, m6, m7, m8)

    for h in range(_NCH):
        @pl.when(sid == 0)
        def _():
            pltpu.sync_copy(z_hbm, acc_shared)

        plsc.subcore_barrier()

        @pl.loop(0, E_PER_W // _K5_C)
        def _(i):
            e0 = base + i * _K5_C
            pltpu.sync_copy(msgs[h].at[pl.ds(e0, _K5_C)], buf_ref)
            pltpu.sync_copy(dst_hbm.at[pl.ds(e0, _K5_C)], idx_ref)
            pltpu.sync_copy(buf_ref, acc_shared.at[idx_ref], add=True)

        plsc.subcore_barrier()

        @pl.when(sid == 0)
        def _():
            pltpu.sync_copy(acc_shared, mpart_hbm.at[cid, h])

        plsc.subcore_barrier()


def _k5(msgs, dst, zerosN):
    f = pl.kernel(
        _k5_body,
        out_type=jax.ShapeDtypeStruct((NC, _NCH, N, HID), jnp.float32),
        mesh=_mesh,
        scratch_types=[
            pltpu.VMEM((_K5_C, HID), jnp.float32),
            pltpu.VMEM((_K5_C,), jnp.int32),
            pltpu.VMEM_SHARED((N, HID), jnp.float32),
        ],
    )
    return f(*msgs, dst, zerosN)


def _k6_body(mpart_ref, bias_ref, out_ref):
    m = mpart_ref[...]
    den = m[0, H] + m[1, H]
    acc = None
    for h in range(H):
        wh = 1.0 / (den[:, h:h + 1] + 1e-16)
        t = (m[0, h] + m[1, h]) * wh
        acc = t if acc is None else acc + t
    out_ref[...] = jnp.tanh(acc * (1.0 / H) + bias_ref[...])


def _k6(mpart, bias):
    R = 1000
    return pl.pallas_call(
        _k6_body,
        grid=(N // R,),
        in_specs=[
            pl.BlockSpec((NC, _NCH, R, HID), lambda i: (0, 0, i, 0)),
            pl.BlockSpec((1, HID), lambda i: (0, 0)),
        ],
        out_specs=pl.BlockSpec((R, HID), lambda i: (i, 0)),
        out_shape=jax.ShapeDtypeStruct((N, HID), jnp.float32),
    )(mpart, bias.reshape(1, -1))


def _k7_body(h_ref, batch_ref, wlin_ref, blin_ref, out_ref):
    b = batch_ref[...]
    gid = lax.broadcasted_iota(jnp.int32, (G, N), 0)
    onehot = (gid == b).astype(jnp.float32)
    counts = jnp.sum(onehot, axis=1, keepdims=True)
    gsum = jnp.dot(onehot, h_ref[...], preferred_element_type=jnp.float32)
    gmean = gsum / jnp.maximum(counts, 1.0)
    out_ref[...] = jnp.tanh(
        jnp.dot(gmean, wlin_ref[...], preferred_element_type=jnp.float32)
        + blin_ref[...]
    )


def _k7(h_nodes, batch, W_lin, b_lin):
    return pl.pallas_call(
        _k7_body,
        grid=(1,),
        in_specs=[
            pl.BlockSpec((N, HID), lambda i: (0, 0)),
            pl.BlockSpec((1, N), lambda i: (0, 0)),
            pl.BlockSpec((HID, D_OUT), lambda i: (0, 0)),
            pl.BlockSpec((1, D_OUT), lambda i: (0, 0)),
        ],
        out_specs=pl.BlockSpec((G, D_OUT), lambda i: (0, 0)),
        out_shape=jax.ShapeDtypeStruct((G, D_OUT), jnp.float32),
    )(h_nodes, batch.reshape(1, -1), W_lin, b_lin.reshape(1, -1))


def kernel(x, edge_index, edge_features, batch, W_l, b_l, W_r, b_r, W_e, att,
           bias, W_lin, b_lin):
    src = edge_index[0]
    dst = edge_index[1]
    xl, xr = _k1(x, W_l, b_l, W_r, b_r)
    xj, xi = _k2(xl, xr, src, dst)
    ex, *msgs = _k3(xj, xi, edge_features, W_e, att)
    mpart = _k5(msgs + [ex], dst, jnp.zeros((N, HID), jnp.float32))
    h_nodes = _k6(mpart, bias)
    graph_embeddings = _k7(h_nodes, batch, W_lin, b_lin)
    return graph_embeddings, h_nodes


# speedup vs baseline: 14.8279x; 1.4868x over previous
import functools

import jax
import jax.numpy as jnp
from jax import lax
from jax.experimental import pallas as pl
from jax.experimental.pallas import tpu as pltpu
from jax.experimental.pallas import tpu_sc as plsc

N = 10000
E = 320000
D_IN = 128
HID = 128
H = 8
D_OUT = 128
G = 64
NEG_SLOPE = 0.2

NC = 2
NS = 16
NW = NC * NS
E_PER_W = E // NW

_mesh = plsc.VectorSubcoreMesh(
    core_axis_name="c", subcore_axis_name="s", num_cores=NC, num_subcores=NS
)


def _pack_bf16(v):
    bits = lax.bitcast_convert_type(v, jnp.int32)
    r16 = (bits + 0x7FFF + ((bits >> 16) & 1)) >> 16
    a = r16[:, :512]
    b = r16[:, 512:]
    return (b << 16) | (a & 0xFFFF)


def _unpack_bf16(p):
    lo = lax.bitcast_convert_type(p << 16, jnp.float32)
    hi = lax.bitcast_convert_type(p & jnp.int32(-65536), jnp.float32)
    return jnp.concatenate([lo, hi], axis=1)


def _k1_body(x_ref, wl_ref, bl_ref, wr_ref, br_ref, xl_ref, xr_ref):
    x = x_ref[...].astype(jnp.bfloat16)
    xl = jnp.dot(x, wl_ref[...].astype(jnp.bfloat16),
                 preferred_element_type=jnp.float32) + bl_ref[...]
    xr = jnp.dot(x, wr_ref[...].astype(jnp.bfloat16),
                 preferred_element_type=jnp.float32) + br_ref[...]
    xl_ref[...] = _pack_bf16(xl)
    xr_ref[...] = _pack_bf16(xr)


def _k1(x, W_l, b_l, W_r, b_r):
    R = 1000
    return pl.pallas_call(
        _k1_body,
        grid=(N // R,),
        in_specs=[
            pl.BlockSpec((R, D_IN), lambda i: (i, 0)),
            pl.BlockSpec((D_IN, H * HID), lambda i: (0, 0)),
            pl.BlockSpec((1, H * HID), lambda i: (0, 0)),
            pl.BlockSpec((D_IN, H * HID), lambda i: (0, 0)),
            pl.BlockSpec((1, H * HID), lambda i: (0, 0)),
        ],
        out_specs=[
            pl.BlockSpec((R, H * HID // 2), lambda i: (i, 0)),
            pl.BlockSpec((R, H * HID // 2), lambda i: (i, 0)),
        ],
        out_shape=[
            jax.ShapeDtypeStruct((N, H * HID // 2), jnp.int32),
            jax.ShapeDtypeStruct((N, H * HID // 2), jnp.int32),
        ],
    )(x, W_l, b_l.reshape(1, -1), W_r, b_r.reshape(1, -1))


_K2_C = 80
_K2_CPT = E // NS // _K2_C


def _k2_body(xl_hbm, xr_hbm, src2_hbm, dst2_hbm, xj_hbm, xi_hbm,
             idx2_ref, a_ref, b_ref, sga, sgb, swa, swb):
    cid = lax.axis_index("c")
    sid = lax.axis_index("s")

    def run(table, idx_src, out):
        cb = sid * _K2_CPT
        pltpu.sync_copy(idx_src.at[sid], idx2_ref)

        @pl.loop(0, _K2_CPT // 2)
        def _(g):
            @pl.when(g > 0)
            def _():
                pltpu.make_async_copy(a_ref, out.at[pl.ds(0, _K2_C)], swa).wait()
                pltpu.make_async_copy(b_ref, out.at[pl.ds(0, _K2_C)], swb).wait()

            ha = pltpu.async_copy(table.at[idx2_ref.at[2 * g]], a_ref, sga)
            hb = pltpu.async_copy(table.at[idx2_ref.at[2 * g + 1]], b_ref, sgb)
            e0 = (cb + 2 * g) * _K2_C
            ha.wait()
            pltpu.async_copy(a_ref, out.at[pl.ds(e0, _K2_C)], swa)
            hb.wait()
            pltpu.async_copy(b_ref, out.at[pl.ds(e0 + _K2_C, _K2_C)], swb)

        pltpu.make_async_copy(a_ref, out.at[pl.ds(0, _K2_C)], swa).wait()
        pltpu.make_async_copy(b_ref, out.at[pl.ds(0, _K2_C)], swb).wait()

    @pl.when(cid == 0)
    def _():
        run(xl_hbm, src2_hbm, xj_hbm)

    @pl.when(cid == 1)
    def _():
        run(xr_hbm, dst2_hbm, xi_hbm)


def _k2(xl, xr, src2, dst2):
    f = pl.kernel(
        _k2_body,
        out_type=[
            jax.ShapeDtypeStruct((E, H * HID // 2), jnp.int32),
            jax.ShapeDtypeStruct((E, H * HID // 2), jnp.int32),
        ],
        mesh=_mesh,
        scratch_types=[
            pltpu.VMEM((_K2_CPT, _K2_C), jnp.int32),
            pltpu.VMEM((_K2_C, H * HID // 2), jnp.int32),
            pltpu.VMEM((_K2_C, H * HID // 2), jnp.int32),
            pltpu.SemaphoreType.DMA,
            pltpu.SemaphoreType.DMA,
            pltpu.SemaphoreType.DMA,
            pltpu.SemaphoreType.DMA,
        ],
    )
    return f(xl, xr, src2, dst2)


def _k3_body(xj_ref, xi_ref, ef_ref, we_ref, att_ref, ex_ref, *msg_refs):
    ee = jnp.dot(ef_ref[...].astype(jnp.bfloat16),
                 we_ref[...].astype(jnp.bfloat16),
                 preferred_element_type=jnp.float32)
    xj = _unpack_bf16(xj_ref[...])
    u = xj + _unpack_bf16(xi_ref[...]) + ee
    z = jnp.where(u >= 0.0, u, NEG_SLOPE * u)
    att = att_ref[...]
    exs = []
    for h in range(H):
        zh = z[:, h * HID:(h + 1) * HID]
        alpha_h = jnp.sum(zh * att[h:h + 1, :], axis=1, keepdims=True)
        ex_h = jnp.exp(alpha_h)
        exs.append(ex_h)
        msg_refs[h][...] = xj[:, h * HID:(h + 1) * HID] * ex_h
    ex = jnp.concatenate(exs, axis=1)
    ex_ref[...] = jnp.concatenate(
        [ex, jnp.zeros((ex.shape[0], HID - H), jnp.float32)], axis=1)


def _k3(xj, xi, ef, W_e, att):
    BE = 1000
    return pl.pallas_call(
        _k3_body,
        grid=(E // BE,),
        in_specs=[
            pl.BlockSpec((BE, H * HID // 2), lambda i: (i, 0)),
            pl.BlockSpec((BE, H * HID // 2), lambda i: (i, 0)),
            pl.BlockSpec((BE, D_IN), lambda i: (i, 0)),
            pl.BlockSpec((D_IN, H * HID), lambda i: (0, 0)),
            pl.BlockSpec((H, HID), lambda i: (0, 0)),
        ],
        out_specs=[pl.BlockSpec((BE, HID), lambda i: (i, 0))]
        + [pl.BlockSpec((BE, HID), lambda i: (i, 0)) for _ in range(H)],
        out_shape=[jax.ShapeDtypeStruct((E, HID), jnp.float32)]
        + [jax.ShapeDtypeStruct((E, HID), jnp.float32) for _ in range(H)],
    )(xj, xi, ef, W_e, att)


_K5_C = 80
_NCH = H + 1


_K5_CPT = E_PER_W // _K5_C
_NROWS = N // NS


def _k5_body(m0, m1, m2, m3, m4, m5, m6, m7, m8, dst2_hbm, z_hbm, mpart_hbm,
             idx2_ref, a_ref, b_ref, acc_shared, sra, srb):
    cid = lax.axis_index("c")
    sid = lax.axis_index("s")
    wid = cid * NS + sid
    base = wid * E_PER_W
    msgs = (m0, m1, m2, m3, m4, m5, m6, m7, m8)
    last = _K5_CPT - 1

    pltpu.sync_copy(dst2_hbm.at[wid], idx2_ref)

    for h in range(_NCH):
        msg = msgs[h]

        @pl.when(sid == 0)
        def _():
            pltpu.sync_copy(z_hbm, acc_shared)

        plsc.subcore_barrier()

        pltpu.async_copy(msg.at[pl.ds(base, _K5_C)], a_ref, sra)
        pltpu.async_copy(msg.at[pl.ds(base + _K5_C, _K5_C)], b_ref, srb)

        @pl.loop(0, (_K5_CPT - 1) // 2)
        def _(g):
            pltpu.make_async_copy(msg.at[pl.ds(base, _K5_C)], a_ref, sra).wait()
            pltpu.sync_copy(a_ref, acc_shared.at[idx2_ref.at[2 * g]], add=True)

            @pl.when(2 * g + 2 <= last)
            def _():
                pltpu.async_copy(
                    msg.at[pl.ds(base + (2 * g + 2) * _K5_C, _K5_C)], a_ref, sra)

            pltpu.make_async_copy(msg.at[pl.ds(base, _K5_C)], b_ref, srb).wait()
            pltpu.sync_copy(b_ref, acc_shared.at[idx2_ref.at[2 * g + 1]], add=True)

            @pl.when(2 * g + 3 <= last)
            def _():
                pltpu.async_copy(
                    msg.at[pl.ds(base + (2 * g + 3) * _K5_C, _K5_C)], b_ref, srb)

        pltpu.make_async_copy(msg.at[pl.ds(base, _K5_C)], a_ref, sra).wait()
        pltpu.sync_copy(a_ref, acc_shared.at[idx2_ref.at[last]], add=True)

        plsc.subcore_barrier()

        @pl.when(sid == 0)
        def _():
            pltpu.sync_copy(acc_shared, mpart_hbm.at[cid, h])

        plsc.subcore_barrier()


def _k5(msgs, dst2, zerosN):
    f = pl.kernel(
        _k5_body,
        out_type=jax.ShapeDtypeStruct((NC, _NCH, N, HID), jnp.float32),
        mesh=_mesh,
        scratch_types=[
            pltpu.VMEM((_K5_CPT, _K5_C), jnp.int32),
            pltpu.VMEM((_K5_C, HID), jnp.float32),
            pltpu.VMEM((_K5_C, HID), jnp.float32),
            pltpu.VMEM_SHARED((N, HID), jnp.float32),
            pltpu.SemaphoreType.DMA,
            pltpu.SemaphoreType.DMA,
        ],
    )
    return f(*msgs, dst2, zerosN)


def _k6_body(mpart_ref, bias_ref, out_ref):
    m = mpart_ref[...]
    den = m[0, H] + m[1, H]
    acc = None
    for h in range(H):
        wh = 1.0 / (den[:, h:h + 1] + 1e-16)
        t = (m[0, h] + m[1, h]) * wh
        acc = t if acc is None else acc + t
    out_ref[...] = jnp.tanh(acc * (1.0 / H) + bias_ref[...])


def _k6(mpart, bias):
    R = 1000
    return pl.pallas_call(
        _k6_body,
        grid=(N // R,),
        in_specs=[
            pl.BlockSpec((NC, _NCH, R, HID), lambda i: (0, 0, i, 0)),
            pl.BlockSpec((1, HID), lambda i: (0, 0)),
        ],
        out_specs=pl.BlockSpec((R, HID), lambda i: (i, 0)),
        out_shape=jax.ShapeDtypeStruct((N, HID), jnp.float32),
    )(mpart, bias.reshape(1, -1))


def _k7_body(h_ref, batch_ref, wlin_ref, blin_ref, out_ref):
    b = batch_ref[...]
    gid = lax.broadcasted_iota(jnp.int32, (G, N), 0)
    onehot = (gid == b).astype(jnp.float32)
    counts = jnp.sum(onehot, axis=1, keepdims=True)
    gsum = jnp.dot(onehot, h_ref[...], preferred_element_type=jnp.float32)
    gmean = gsum / jnp.maximum(counts, 1.0)
    out_ref[...] = jnp.tanh(
        jnp.dot(gmean, wlin_ref[...], preferred_element_type=jnp.float32)
        + blin_ref[...]
    )


def _k7(h_nodes, batch, W_lin, b_lin):
    return pl.pallas_call(
        _k7_body,
        grid=(1,),
        in_specs=[
            pl.BlockSpec((N, HID), lambda i: (0, 0)),
            pl.BlockSpec((1, N), lambda i: (0, 0)),
            pl.BlockSpec((HID, D_OUT), lambda i: (0, 0)),
            pl.BlockSpec((1, D_OUT), lambda i: (0, 0)),
        ],
        out_specs=pl.BlockSpec((G, D_OUT), lambda i: (0, 0)),
        out_shape=jax.ShapeDtypeStruct((G, D_OUT), jnp.float32),
    )(h_nodes, batch.reshape(1, -1), W_lin, b_lin.reshape(1, -1))


def kernel(x, edge_index, edge_features, batch, W_l, b_l, W_r, b_r, W_e, att,
           bias, W_lin, b_lin):
    src3 = edge_index[0].reshape(NS, _K2_CPT, _K2_C)
    dst3 = edge_index[1].reshape(NS, _K2_CPT, _K2_C)
    dst3w = edge_index[1].reshape(NW, _K5_CPT, _K5_C)
    xl, xr = _k1(x, W_l, b_l, W_r, b_r)
    xj, xi = _k2(xl, xr, src3, dst3)
    ex, *msgs = _k3(xj, xi, edge_features, W_e, att)
    mpart = _k5(msgs + [ex], dst3w, jnp.zeros((N, HID), jnp.float32))
    h_nodes = _k6(mpart, bias)
    graph_embeddings = _k7(h_nodes, batch, W_lin, b_lin)
    return graph_embeddings, h_nodes
